# Initial kernel scaffold; baseline (speedup 1.0000x reference)
import jax
import jax.numpy as jnp
from jax.experimental import pallas as pl


def kernel(x_A, x_B, edge_index_A_to_B, edge_index_B_to_A, batch_A, batch_B, batch_size, W_rel_l0_AB, b_rel_l0_AB, W_root_l0_AB, W_rel_l0_BA, b_rel_l0_BA, W_root_l0_BA, W_rel_l1_AB, b_rel_l1_AB, W_root_l1_AB, W_rel_l1_BA, b_rel_l1_BA, W_root_l1_BA, W_cls, b_cls):
    raise NotImplementedError("write your pallas kernel here")


# speedup vs baseline: 3.9571x; 3.9571x over previous
import functools

import jax
import jax.numpy as jnp
from jax import lax
from jax.experimental import pallas as pl
from jax.experimental.pallas import tpu as pltpu
from jax.experimental.pallas import tpu_sc as plsc

N = 10000
E = 320000
H = 128
B = 64
C = 32

NC = 2
NS = 16
NW = NC * NS
EPW = E // NW
CH = 80
NCHUNK = EPW // CH
RPT = 624
TAIL = N - NS * RPT
ZR = 312

_F32 = jnp.float32
_HIGH = jax.lax.Precision.HIGHEST


def _dot(a, b):
    return jnp.dot(a, b, preferred_element_type=_F32, precision=_HIGH)


def _counts_body(dab_hbm, dba_hbm, out_hbm, acc_a, acc_b, e0b, zb2, dbuf):
    c = lax.axis_index("c")
    s = lax.axis_index("s")
    wid = s * NC + c
    zero16 = jnp.zeros((16,), _F32)
    e016 = jnp.where(lax.iota(jnp.int32, 16) == 0, 1.0, 0.0).astype(_F32)

    def init_e(i, carry):
        e0b[i] = e016
        return carry

    lax.fori_loop(0, CH, init_e, 0)

    def init_z(i, carry):
        zb2[i] = zero16
        return carry

    lax.fori_loop(0, RPT, init_z, 0)
    pltpu.sync_copy(zb2, acc_a.at[pl.ds(s * RPT, RPT)])
    pltpu.sync_copy(zb2, acc_b.at[pl.ds(s * RPT, RPT)])

    @pl.when(s == NS - 1)
    def _():
        pltpu.sync_copy(zb2.at[pl.ds(0, TAIL)], acc_a.at[pl.ds(NS * RPT, TAIL)])
        pltpu.sync_copy(zb2.at[pl.ds(0, TAIL)], acc_b.at[pl.ds(NS * RPT, TAIL)])

    plsc.subcore_barrier()

    for t in range(2):
        d_hbm = dab_hbm if t == 0 else dba_hbm
        acc = acc_b if t == 0 else acc_a

        def body(i, carry):
            pltpu.sync_copy(d_hbm.at[pl.ds(wid * EPW + i * CH, CH)], dbuf)
            pltpu.sync_copy(e0b, acc.at[dbuf], add=True)
            return carry

        lax.fori_loop(0, NCHUNK, body, 0)

    plsc.subcore_barrier()
    for t in range(2):
        acc = acc_b if t == 0 else acc_a
        pltpu.sync_copy(acc.at[pl.ds(s * RPT, RPT)], zb2)
        pltpu.sync_copy(zb2, out_hbm.at[t, c, pl.ds(s * RPT, RPT)])

        @pl.when(s == NS - 1)
        def _():
            pltpu.sync_copy(acc.at[pl.ds(NS * RPT, TAIL)], zb2.at[pl.ds(0, TAIL)])
            pltpu.sync_copy(zb2.at[pl.ds(0, TAIL)],
                            out_hbm.at[t, c, pl.ds(NS * RPT, TAIL)])


def _sc_counts(dst_ab, dst_ba):
    mesh = plsc.VectorSubcoreMesh(core_axis_name="c", subcore_axis_name="s")
    f = pl.kernel(
        _counts_body,
        out_type=jax.ShapeDtypeStruct((2, NC, N, 16), _F32),
        mesh=mesh,
        scratch_types=[
            pltpu.VMEM_SHARED((N, 16), _F32),
            pltpu.VMEM_SHARED((N, 16), _F32),
            pltpu.VMEM((CH, 16), _F32),
            pltpu.VMEM((RPT, 16), _F32),
            pltpu.VMEM((CH,), jnp.int32),
        ],
    )
    return f(dst_ab, dst_ba)


def _scatter_body(y_hbm, src_hbm, dst_hbm, out_hbm, acc, zbuf, rows, sidx,
                  didx, sem):
    c = lax.axis_index("c")
    s = lax.axis_index("s")
    wid = s * NC + c
    zero16 = jnp.zeros((16,), _F32)

    def zb(i, carry):
        for j in range(H // 16):
            zbuf[i, pl.ds(j * 16, 16)] = zero16
        return carry

    lax.fori_loop(0, ZR, zb, 0)
    for i in range(RPT // ZR):
        pltpu.sync_copy(zbuf, acc.at[pl.ds(s * RPT + i * ZR, ZR)])

    @pl.when(s == NS - 1)
    def _():
        pltpu.sync_copy(zbuf.at[pl.ds(0, TAIL)], acc.at[pl.ds(NS * RPT, TAIL)])

    plsc.subcore_barrier()

    def body(i, carry):
        base = wid * EPW + i * CH
        pltpu.sync_copy(src_hbm.at[pl.ds(base, CH)], sidx)
        pltpu.async_copy(y_hbm.at[sidx], rows, sem).wait()
        pltpu.sync_copy(dst_hbm.at[pl.ds(base, CH)], didx)
        pltpu.sync_copy(rows, acc.at[didx], add=True)
        return carry

    lax.fori_loop(0, NCHUNK, body, 0)
    plsc.subcore_barrier()
    for i in range(RPT // ZR):
        off = s * RPT + i * ZR
        pltpu.sync_copy(acc.at[pl.ds(off, ZR)], zbuf)
        pltpu.sync_copy(zbuf, out_hbm.at[c, pl.ds(off, ZR)])

    @pl.when(s == NS - 1)
    def _():
        pltpu.sync_copy(acc.at[pl.ds(NS * RPT, TAIL)], zbuf.at[pl.ds(0, TAIL)])
        pltpu.sync_copy(zbuf.at[pl.ds(0, TAIL)],
                        out_hbm.at[c, pl.ds(NS * RPT, TAIL)])


def _sc_scatter(y, src, dst):
    mesh = plsc.VectorSubcoreMesh(core_axis_name="c", subcore_axis_name="s")
    f = pl.kernel(
        _scatter_body,
        out_type=jax.ShapeDtypeStruct((NC, N, H), _F32),
        mesh=mesh,
        scratch_types=[
            pltpu.VMEM_SHARED((N, H), _F32),
            pltpu.VMEM((ZR, H), _F32),
            pltpu.VMEM((CH, H), _F32),
            pltpu.VMEM((CH,), jnp.int32),
            pltpu.VMEM((CH,), jnp.int32),
            pltpu.SemaphoreType.DMA,
        ],
    )
    return f(y, src, dst)


def _inv_body(cab_ref, cba_ref, invb_ref, inva_ref):
    cb = jnp.sum(cab_ref[...], axis=(0, 2))
    ca = jnp.sum(cba_ref[...], axis=(0, 2))
    invb_ref[...] = jnp.broadcast_to(
        (1.0 / jnp.clip(cb, 1.0, None))[:, None], (N, H))
    inva_ref[...] = jnp.broadcast_to(
        (1.0 / jnp.clip(ca, 1.0, None))[:, None], (N, H))


def _tc_inv(cnt_ab, cnt_ba):
    return pl.pallas_call(
        _inv_body,
        out_shape=[jax.ShapeDtypeStruct((N, H), _F32),
                   jax.ShapeDtypeStruct((N, H), _F32)],
    )(cnt_ab, cnt_ba)


_RB = 1000
_GRID = N // _RB


def _dense_body(ha_ref, hb_ref, wr_ab_ref, wo_ba_ref, br_ba_ref, wr_ba_ref,
                wo_ab_ref, br_ab_ref, ya_ref, za_ref, yb_ref, zb_ref):
    a = ha_ref[...]
    b = hb_ref[...]
    ya_ref[...] = _dot(a, wr_ab_ref[...])
    za_ref[...] = _dot(a, wo_ba_ref[...]) + br_ba_ref[...]
    yb_ref[...] = _dot(b, wr_ba_ref[...])
    zb_ref[...] = _dot(b, wo_ab_ref[...]) + br_ab_ref[...]


def _tc_dense(ha, hb, wr_ab, wo_ba, br_ba, wr_ba, wo_ab, br_ab):
    row = pl.BlockSpec((_RB, H), lambda i: (i, 0))
    w = pl.BlockSpec((H, H), lambda i: (0, 0))
    bias = pl.BlockSpec((1, H), lambda i: (0, 0))
    return pl.pallas_call(
        _dense_body,
        grid=(_GRID,),
        in_specs=[row, row, w, w, bias, w, w, bias],
        out_specs=[row, row, row, row],
        out_shape=[jax.ShapeDtypeStruct((N, H), _F32)] * 4,
    )(ha, hb, wr_ab, wo_ba, br_ba.reshape(1, H), wr_ba, wo_ab,
      br_ab.reshape(1, H))


def _combine_body(sa0_ref, sa1_ref, za_ref, inva_ref, sb0_ref, sb1_ref,
                  zb_ref, invb_ref, ha_ref, hb_ref):
    ha_ref[...] = jnp.maximum(
        (sa0_ref[...] + sa1_ref[...]) * inva_ref[...] + za_ref[...], 0.0)
    hb_ref[...] = jnp.maximum(
        (sb0_ref[...] + sb1_ref[...]) * invb_ref[...] + zb_ref[...], 0.0)


def _tc_combine(sa, za, inva, sb, zb, invb):
    row = pl.BlockSpec((_RB, H), lambda i: (i, 0))
    return pl.pallas_call(
        _combine_body,
        grid=(_GRID,),
        in_specs=[row] * 8,
        out_specs=[row, row],
        out_shape=[jax.ShapeDtypeStruct((N, H), _F32)] * 2,
    )(sa[0], sa[1], za, inva, sb[0], sb[1], zb, invb)


def _pool_body(pa_ref, pb_ref, ha_ref, hb_ref, wc_ref, bc_ref, out_ref):
    pa = pa_ref[...]
    pb = pb_ref[...]
    sa = _dot(pa, ha_ref[...])
    sb = _dot(pb, hb_ref[...])
    ca = jnp.sum(pa, axis=1)
    cb = jnp.sum(pb, axis=1)
    ma = sa * (1.0 / jnp.clip(ca, 1.0, None))[:, None]
    mb = sb * (1.0 / jnp.clip(cb, 1.0, None))[:, None]
    pooled = jnp.concatenate([ma, mb], axis=1)
    out_ref[...] = _dot(pooled, wc_ref[...]) + bc_ref[...]


def _tc_pool(pa, pb, ha, hb, w_cls, b_cls):
    return pl.pallas_call(
        _pool_body,
        out_shape=jax.ShapeDtypeStruct((B, C), _F32),
    )(pa, pb, ha, hb, w_cls, b_cls.reshape(1, C))


def kernel(x_A, x_B, edge_index_A_to_B, edge_index_B_to_A, batch_A, batch_B,
           batch_size, W_rel_l0_AB, b_rel_l0_AB, W_root_l0_AB, W_rel_l0_BA,
           b_rel_l0_BA, W_root_l0_BA, W_rel_l1_AB, b_rel_l1_AB, W_root_l1_AB,
           W_rel_l1_BA, b_rel_l1_BA, W_root_l1_BA, W_cls, b_cls):
    src_ab = edge_index_A_to_B[0]
    dst_ab = edge_index_A_to_B[1]
    src_ba = edge_index_B_to_A[0]
    dst_ba = edge_index_B_to_A[1]

    cnt = _sc_counts(dst_ab, dst_ba)
    invb, inva = _tc_inv(cnt[0], cnt[1])

    layers = [
        (W_rel_l0_AB, b_rel_l0_AB, W_root_l0_AB,
         W_rel_l0_BA, b_rel_l0_BA, W_root_l0_BA),
        (W_rel_l1_AB, b_rel_l1_AB, W_root_l1_AB,
         W_rel_l1_BA, b_rel_l1_BA, W_root_l1_BA),
    ]
    ha, hb = x_A, x_B
    for wr_ab, br_ab, wo_ab, wr_ba, br_ba, wo_ba in layers:
        ya, za, yb, zb = _tc_dense(ha, hb, wr_ab, wo_ba, br_ba, wr_ba,
                                   wo_ab, br_ab)
        sb = _sc_scatter(ya, src_ab, dst_ab)
        sa = _sc_scatter(yb, src_ba, dst_ba)
        ha, hb = _tc_combine(sa, za, inva, sb, zb, invb)

    ids = jnp.arange(B, dtype=jnp.int32)[:, None]
    pa = (batch_A[None, :] == ids).astype(_F32)
    pb = (batch_B[None, :] == ids).astype(_F32)
    out = _tc_pool(pa, pb, ha, hb, W_cls, b_cls)
    zero_dep = (jnp.asarray(batch_size, jnp.int32) - jnp.int32(B)).astype(
        W_cls.dtype)
    return out + zero_dep
